# Initial kernel scaffold; baseline (speedup 1.0000x reference)
import jax
import jax.numpy as jnp
from jax.experimental import pallas as pl


def kernel(elements, coords, nl, A, B, p, q, sigma, gamma, cutoff, lam, cos_beta0, cutoff_jk):
    raise NotImplementedError("write your pallas kernel here")


# speedup vs baseline: 13.2361x; 13.2361x over previous
import functools

import jax
import jax.numpy as jnp
import numpy as np
from jax import lax
from jax.experimental import pallas as pl
from jax.experimental.pallas import tpu as pltpu
from jax.experimental.pallas import tpu_sc as plsc

N_ATOMS = 10000
D_NBR = 32
NCOL = D_NBR + 1
ROWW = 16
GCOLS = NCOL * ROWW
NPAIR = (D_NBR * (D_NBR - 1)) // 2
PPAD = 512
BA = 200


def _build_consts():
    jj, kk = np.triu_indices(D_NBR, k=1)
    m1 = np.zeros((GCOLS, 96), np.float32)
    for c in range(3):
        for j in range(D_NBR):
            m1[ROWW * (j + 1) + c, 32 * c + j] = 1.0
            m1[c, 32 * c + j] = -1.0
    mjk = np.zeros((D_NBR, 2 * PPAD), np.float32)
    for p in range(NPAIR):
        mjk[jj[p], p] = 1.0
        mjk[kk[p], PPAD + p] = 1.0
    dp = np.zeros((D_NBR, PPAD), np.float32)
    for p in range(NPAIR):
        dp[kk[p], p] += 1.0
        dp[jj[p], p] -= 1.0
    return jnp.asarray(m1), jnp.asarray(mjk), jnp.asarray(dp)


def _quad_coeffs(t):
    c1 = (4.0 * t[1] - 3.0 * t[0] - t[2]) * 0.5
    c2 = (t[2] + t[0] - 2.0 * t[1]) * 0.5
    return t[0], c1, c2


def _sc_gather(tab_flat, idx_flat, n_atoms, pw, cs):
    mesh = plsc.VectorSubcoreMesh(core_axis_name="c", subcore_axis_name="s")
    nchunk = pw // cs
    nstep = cs // 16

    @functools.partial(
        pl.kernel,
        mesh=mesh,
        compiler_params=pltpu.CompilerParams(needs_layout_passes=False),
        out_type=jax.ShapeDtypeStruct((32 * pw * ROWW,), jnp.float32),
        scratch_types=[
            pltpu.VMEM((n_atoms * 4,), jnp.float32),
            pltpu.VMEM((pw,), jnp.int32),
            pltpu.VMEM((cs * ROWW,), jnp.float32),
        ],
    )
    def k(tab_hbm, idx_hbm, out_hbm, tab_v, idx_v, stage_v):
        wid = lax.axis_index("s") * 2 + lax.axis_index("c")
        pltpu.sync_copy(tab_hbm, tab_v)
        pltpu.sync_copy(idx_hbm.at[pl.ds(wid * pw, pw)], idx_v)
        lane = lax.iota(jnp.int32, 16)

        def chunk(ci, _):
            def step(s, _):
                atom = idx_v[pl.ds(ci * cs + s * 16, 16)]
                src = atom * 4
                dst = (s * 16 + lane) * ROWW
                for c in range(3):
                    v = plsc.load_gather(tab_v, [src + c])
                    plsc.store_scatter(stage_v, [dst + c], v)
                return 0

            lax.fori_loop(0, nstep, step, 0, unroll=4)
            pltpu.sync_copy(
                stage_v,
                out_hbm.at[pl.ds((wid * pw + ci * cs) * ROWW, cs * ROWW)],
            )
            return 0

        lax.fori_loop(0, nchunk, chunk, 0, unroll=False)

    return k(tab_flat, idx_flat)


def _dot(a, b):
    return lax.dot_general(
        a, b, (((1,), (0,)), ((), ())),
        precision=lax.Precision.HIGHEST,
        preferred_element_type=jnp.float32,
    )


def _energy_body(params_ref, g_ref, ei_ref, en_ref, m1_ref, mjk_ref, dp_ref,
                 out_ref):
    i = pl.program_id(0)
    G = g_ref[...]
    ei = ei_ref[...]
    en = en_ref[...]

    R = _dot(G, m1_ref[...])
    Rx = R[:, 0:32]
    Ry = R[:, 32:64]
    Rz = R[:, 64:96]
    rij2 = Rx * Rx + Ry * Ry + Rz * Rz
    rij = jnp.sqrt(rij2)

    def q3(base, s, ssq):
        return (params_ref[base] + params_ref[base + 1] * s
                + params_ref[base + 2] * ssq)

    s2 = ei + en
    s2q = s2 * s2
    A_ij = q3(0, s2, s2q)
    B_ij = q3(3, s2, s2q)
    p_ij = q3(6, s2, s2q)
    q_ij = q3(9, s2, s2q)
    sig_ij = q3(12, s2, s2q)
    cut_ij = q3(18, s2, s2q)
    mask2 = rij < cut_ij
    safe_r = jnp.where(mask2, rij, 1.0)
    log_sr = jnp.log(sig_ij / safe_r)
    Bpq = B_ij * jnp.exp(p_ij * log_sr) - jnp.exp(q_ij * log_sr)
    denom2 = jnp.where(mask2, rij - cut_ij, -1.0)
    E2 = jnp.where(mask2, A_ij * Bpq * jnp.exp(sig_ij / denom2), 0.0)
    esum = 0.5 * jnp.sum(E2)

    MJK = mjk_ref[...]
    RP = _dot(rij, MJK)
    EP = _dot(en, MJK)
    rij_p = RP[:, 0:PPAD]
    rik_p = RP[:, PPAD:]
    ej = EP[:, 0:PPAD]
    ek = EP[:, PPAD:]
    DP = dp_ref[...]
    dx = _dot(Rx, DP)
    dy = _dot(Ry, DP)
    dz = _dot(Rz, DP)
    rjk2 = dx * dx + dy * dy + dz * dz + 1e-20

    s3j = ei + ej
    s3k = ei + ek
    gam_j = q3(15, s3j, s3j * s3j)
    cut_j = q3(18, s3j, s3j * s3j)
    gam_k = q3(15, s3k, s3k * s3k)
    cut_k = q3(18, s3k, s3k * s3k)
    ijk = jnp.clip(2.0 - (ei + ej + ek), 0.0, 1.0)
    lam_v = params_ref[21] + params_ref[22] * ijk
    cb0_v = params_ref[23] + params_ref[24] * ijk
    cjk_v = params_ref[25] + params_ref[26] * ijk

    pad_ok = lax.broadcasted_iota(jnp.int32, (BA, PPAD), 1) < NPAIR
    cond = (ei != ej) & (ej == ek)
    mask3 = (cond & (rij_p <= cut_j) & (rik_p <= cut_k)
             & (rjk2 <= cjk_v * cjk_v) & pad_ok)
    srij = jnp.where(mask3, rij_p, 1.0)
    srik = jnp.where(mask3, rik_p, 1.0)
    cos_b = (rij_p * rij_p + rik_p * rik_p - rjk2) / (2.0 * srij * srik)
    d1 = jnp.where(mask3, rij_p - cut_j, -1.0)
    d2 = jnp.where(mask3, rik_p - cut_k, -1.0)
    dcb = cos_b - cb0_v
    E3 = lam_v * jnp.exp(gam_j / d1 + gam_k / d2) * dcb * dcb
    esum = esum + jnp.sum(jnp.where(mask3, E3, 0.0))

    @pl.when(i == 0)
    def _():
        out_ref[0, 0] = 0.0

    out_ref[0, 0] += esum


_TC_IN_SPECS = [
    pl.BlockSpec(memory_space=pltpu.SMEM),
    pl.BlockSpec((BA, GCOLS), lambda i: (i, 0)),
    pl.BlockSpec((BA, 1), lambda i: (i, 0)),
    pl.BlockSpec((BA, D_NBR), lambda i: (i, 0)),
    pl.BlockSpec((GCOLS, 96), lambda i: (0, 0)),
    pl.BlockSpec((D_NBR, 2 * PPAD), lambda i: (0, 0)),
    pl.BlockSpec((D_NBR, PPAD), lambda i: (0, 0)),
]


def _tc_energy(params, g, ei, en, m1, mjk, dp):
    n = g.shape[0]
    grid = (n // BA,)
    return pl.pallas_call(
        _energy_body,
        grid=grid,
        in_specs=_TC_IN_SPECS,
        out_specs=pl.BlockSpec(memory_space=pltpu.SMEM),
        out_shape=jax.ShapeDtypeStruct((1, 1), jnp.float32),
    )(params, g, ei, en, m1, mjk, dp)


def _pack_params(A, B, p, q, sigma, gamma, cutoff, lam, cos_beta0, cutoff_jk):
    vals = []
    for t in (A, B, p, q, sigma, gamma, cutoff):
        vals.extend(_quad_coeffs(t))
    vals.extend([lam[0], lam[1] - lam[0]])
    vals.extend([cos_beta0[0], cos_beta0[1] - cos_beta0[0]])
    vals.extend([cutoff_jk[0], cutoff_jk[1] - cutoff_jk[0]])
    return jnp.stack(vals).astype(jnp.float32)


def kernel(elements, coords, nl, A, B, p, q, sigma, gamma, cutoff, lam,
           cos_beta0, cutoff_jk):
    n, ncol = nl.shape
    total = n * ncol
    cs = 1152
    pw = -(-total // (32 * cs)) * cs
    padded = 32 * pw

    tab_flat = jnp.zeros((n, 4), jnp.float32).at[:, :3].set(coords).reshape(-1)
    idx = jnp.zeros((padded,), jnp.int32).at[:total].set(nl.reshape(-1))

    g_flat = _sc_gather(tab_flat, idx, n, pw, cs)
    g = g_flat[: total * ROWW].reshape(n, GCOLS)

    ef = elements.astype(jnp.float32)
    ei = ef[:, :1]
    en = ef[:, 1:]
    params = _pack_params(A, B, p, q, sigma, gamma, cutoff, lam, cos_beta0,
                          cutoff_jk)
    m1, mjk, dp = _build_consts()
    out = _tc_energy(params, g, ei, en, m1, mjk, dp)
    return out[0, 0]
